# Initial kernel scaffold; baseline (speedup 1.0000x reference)
import jax
import jax.numpy as jnp
from jax.experimental import pallas as pl


def kernel(x, edges0, edges1, edges2, edges3, params):
    raise NotImplementedError("write your pallas kernel here")


# speedup vs baseline: 18.7116x; 18.7116x over previous
import functools

import jax
import jax.numpy as jnp
from jax import lax
from jax.experimental import pallas as pl
from jax.experimental.pallas import tpu as pltpu
from jax.experimental.pallas import tpu_sc as plsc

N = 10000
NPAD = 10240
D = 128
E = 320000
NUM_MOTIFS = 4
CH = 128
NCHUNK = E // CH
ROWS_PER_TILE = NPAD // 16
F32 = jnp.float32


def _scores_body(h_ref, v_ref, out_ref):
    out_ref[...] = lax.dot_general(
        v_ref[...], h_ref[...], (((0,), (1,)), ((), ())),
        preferred_element_type=F32)


def _tc_scores(h, vsd, blk=1024):
    grid = NPAD // blk
    return pl.pallas_call(
        _scores_body,
        grid=(grid,),
        in_specs=[
            pl.BlockSpec((blk, D), lambda i: (i, 0)),
            pl.BlockSpec((D, 8), lambda i: (0, 0)),
        ],
        out_specs=pl.BlockSpec((8, blk), lambda i: (0, i)),
        out_shape=jax.ShapeDtypeStruct((8, NPAD), F32),
    )(h, vsd)


def _mlp_body(h_ref, acc_ref, den_ref, u0_ref, ucat_ref, c1_ref, w2_ref,
              b2_ref, out_ref):
    t = jnp.dot(h_ref[...], u0_ref[...], preferred_element_type=F32)
    for m in range(NUM_MOTIFS):
        inv = 1.0 / (den_ref[m, :] + 1e-16)
        agg = acc_ref[m] * inv[:, None]
        t += jnp.dot(agg, ucat_ref[m * D:(m + 1) * D, :],
                     preferred_element_type=F32)
    z = jnp.tanh(t + c1_ref[...])
    out_ref[...] = jnp.dot(z, w2_ref[...], preferred_element_type=F32) + b2_ref[...]


def _tc_mlp(h, acc, den, u0, ucat, c1, w2, b2, blk=1024):
    grid = NPAD // blk
    return pl.pallas_call(
        _mlp_body,
        grid=(grid,),
        in_specs=[
            pl.BlockSpec((blk, D), lambda i: (i, 0)),
            pl.BlockSpec((NUM_MOTIFS, blk, D), lambda i: (0, i, 0)),
            pl.BlockSpec((NUM_MOTIFS, blk), lambda i: (0, i)),
            pl.BlockSpec((D, D), lambda i: (0, 0)),
            pl.BlockSpec((NUM_MOTIFS * D, D), lambda i: (0, 0)),
            pl.BlockSpec((1, D), lambda i: (0, 0)),
            pl.BlockSpec((D, D), lambda i: (0, 0)),
            pl.BlockSpec((1, D), lambda i: (0, 0)),
        ],
        out_specs=pl.BlockSpec((blk, D), lambda i: (i, 0)),
        out_shape=jax.ShapeDtypeStruct((NPAD, D), F32),
    )(h, acc, den, u0, ucat, c1, w2, b2)


def _sc_edges_build():
    mesh = plsc.VectorSubcoreMesh(core_axis_name="c", subcore_axis_name="s")

    @functools.partial(
        pl.kernel,
        mesh=mesh,
        compiler_params=pltpu.CompilerParams(needs_layout_passes=False,
                                             use_tc_tiling_on_sc=False),
        out_type=[
            jax.ShapeDtypeStruct((NUM_MOTIFS, NPAD, D), F32),
            jax.ShapeDtypeStruct((NUM_MOTIFS * NPAD,), F32),
        ],
        scratch_types=[
            pltpu.VMEM((N,), F32),
            pltpu.VMEM((N,), F32),
            pltpu.VMEM((CH,), jnp.int32),
            pltpu.VMEM((1, CH), jnp.int32),
            pltpu.VMEM((CH,), F32),
            pltpu.VMEM((CH, D), F32),
            pltpu.VMEM_SHARED((NPAD, D), F32),
            pltpu.VMEM_SHARED((NPAD,), F32),
            pltpu.SemaphoreType.DMA,
        ],
    )
    def sc_edges(s_hbm, edges_hbm, h_hbm, acc_out, den_out,
                 ssrc_v, sdst_v, src_v, dsti_v, ee_v, rows_v,
                 acc_sh, den_sh, sem):
        c = lax.axis_index("c")
        s = lax.axis_index("s")
        base = s * ROWS_PER_TILE
        nchunks = jnp.where(s < NCHUNK % 16, NCHUNK // 16 + 1, NCHUNK // 16)

        for mm in range(2):
            m = 2 * c + mm

            def _zrow(r, _):
                for k in range(D // 16):
                    rows_v[r, pl.ds(16 * k, 16)] = jnp.zeros((16,), F32)
                return 0
            lax.fori_loop(0, CH, _zrow, 0, unroll=False)
            for t in range(CH // 16):
                ee_v[pl.ds(16 * t, 16)] = jnp.zeros((16,), F32)
            for j in range(ROWS_PER_TILE // CH):
                pltpu.sync_copy(rows_v, acc_sh.at[pl.ds(base + CH * j, CH)])
            for j in range(ROWS_PER_TILE // CH):
                pltpu.sync_copy(ee_v, den_sh.at[pl.ds(base + CH * j, CH)])

            soff = pl.multiple_of(2 * m * NPAD, NPAD)
            pltpu.sync_copy(s_hbm.at[pl.ds(soff, N)], ssrc_v)
            pltpu.sync_copy(s_hbm.at[pl.ds(soff + NPAD, N)], sdst_v)

            iota16b = lax.iota(jnp.int32, 16)

            def _redmax(ref):
                def body(i, carry):
                    return jnp.maximum(carry, ref[pl.ds(16 * i, 16)])
                mv = lax.fori_loop(0, N // 16, body,
                                   jnp.full((16,), -3e38, F32))
                for sh in (1, 2, 4, 8):
                    ee_v[pl.ds(0, 16)] = mv
                    pv = plsc.load_gather(ee_v, [iota16b ^ sh])
                    mv = jnp.maximum(mv, pv)
                return mv[0]
            bsum = _redmax(ssrc_v) + _redmax(sdst_v)
            bound = jnp.maximum(bsum, 0.2 * bsum)

            plsc.subcore_barrier()

            def _chunk(i, _):
                off = pl.multiple_of(2 * E * m + (s + 16 * i) * CH, CH)
                pltpu.sync_copy(edges_hbm.at[pl.ds(off, CH)], src_v)
                pltpu.sync_copy(edges_hbm.at[pl.ds(off + E, CH)],
                                dsti_v.at[0])
                pltpu.async_copy(h_hbm.at[src_v], rows_v, sem).wait()
                for t in range(CH // 16):
                    iv = src_v[pl.ds(16 * t, 16)]
                    dv = dsti_v[0, pl.ds(16 * t, 16)]
                    a = plsc.load_gather(ssrc_v, [iv])
                    b = plsc.load_gather(sdst_v, [dv])
                    sv = a + b
                    ev = jnp.maximum(sv, 0.2 * sv) - bound
                    ee_v[pl.ds(16 * t, 16)] = jnp.exp(ev)

                def _srow(t, _):
                    wv = ee_v[pl.ds(16 * t, 16)]
                    rbase = 16 * t
                    for j in range(16):
                        w = wv[j]
                        for k in range(D // 16):
                            rows_v[rbase + j, pl.ds(16 * k, 16)] = (
                                rows_v[rbase + j, pl.ds(16 * k, 16)] * w)
                    return 0
                lax.fori_loop(0, CH // 16, _srow, 0, unroll=False)

                pltpu.sync_copy(rows_v, acc_sh.at[dsti_v.at[0]], add=True)
                pltpu.sync_copy(ee_v, den_sh.at[dsti_v.at[0]], add=True)
                return 0
            lax.fori_loop(0, nchunks, _chunk, 0, unroll=False)

            plsc.subcore_barrier()

            pltpu.sync_copy(acc_sh.at[pl.ds(base, ROWS_PER_TILE)],
                            acc_out.at[m, pl.ds(base, ROWS_PER_TILE)])
            doff = pl.multiple_of(m * NPAD + base, ROWS_PER_TILE)
            pltpu.sync_copy(den_sh.at[pl.ds(base, ROWS_PER_TILE)],
                            den_out.at[pl.ds(doff, ROWS_PER_TILE)])

            plsc.subcore_barrier()

    return sc_edges


_SC_EDGES = None


def _get_sc_edges():
    global _SC_EDGES
    if _SC_EDGES is None:
        _SC_EDGES = _sc_edges_build()
    return _SC_EDGES


def kernel(x, edges0, edges1, edges2, edges3, params):
    edges = jnp.stack([edges0.astype(jnp.int32), edges1.astype(jnp.int32),
                       edges2.astype(jnp.int32),
                       edges3.astype(jnp.int32)]).reshape(-1)
    h = jnp.pad(x.astype(F32), ((0, NPAD - N), (0, 0)))
    sc_edges = _get_sc_edges()

    for l in range(2):
        vcols = []
        ucols = []
        c1 = params[f"mlpb1_{l}"]
        for m in range(NUM_MOTIFS):
            W = params[f"W_{l}_{m}"]
            vcols.append(W @ params[f"as_{l}_{m}"])
            vcols.append(W @ params[f"ad_{l}_{m}"])
            w1m = params[f"mlpW1_{l}"][(m + 1) * D:(m + 2) * D, :]
            ucols.append(W @ w1m)
            c1 = c1 + params[f"b_{l}_{m}"] @ w1m
        vsd = jnp.stack(vcols, axis=1)
        ucat = jnp.concatenate(ucols, axis=0)
        u0 = params[f"mlpW1_{l}"][:D, :]
        w2 = params[f"mlpW2_{l}"]
        b2 = params[f"mlpb2_{l}"][None, :]
        c1 = c1[None, :]

        s_t = _tc_scores(h, vsd).reshape(-1)
        acc, den = sc_edges(s_t, edges, h)
        h = _tc_mlp(h, acc, den.reshape(NUM_MOTIFS, NPAD), u0, ucat, c1,
                    w2, b2)

    return h[:N]


# speedup vs baseline: 26.3682x; 1.4092x over previous
import functools

import jax
import jax.numpy as jnp
from jax import lax
from jax.experimental import pallas as pl
from jax.experimental.pallas import tpu as pltpu
from jax.experimental.pallas import tpu_sc as plsc

N = 10000
NPAD = 10240
D = 128
E = 320000
NUM_MOTIFS = 4
CH = 80
NCHUNK = E // CH
CPT = NCHUNK // 16
ROWS_PER_TILE = NPAD // 16
F32 = jnp.float32


def _scores_body(h_ref, v_ref, out_ref):
    out_ref[...] = lax.dot_general(
        v_ref[...], h_ref[...], (((0,), (1,)), ((), ())),
        preferred_element_type=F32)


def _tc_scores(h, vsd, blk=1024):
    grid = NPAD // blk
    return pl.pallas_call(
        _scores_body,
        grid=(grid,),
        in_specs=[
            pl.BlockSpec((blk, D), lambda i: (i, 0)),
            pl.BlockSpec((D, 8), lambda i: (0, 0)),
        ],
        out_specs=pl.BlockSpec((8, blk), lambda i: (0, i)),
        out_shape=jax.ShapeDtypeStruct((8, NPAD), F32),
    )(h, vsd)


def _mlp_body(h_ref, acc_ref, den_ref, u0_ref, ucat_ref, c1_ref, w2_ref,
              b2_ref, out_ref):
    t = jnp.dot(h_ref[...], u0_ref[...], preferred_element_type=F32)
    for m in range(NUM_MOTIFS):
        inv = 1.0 / (den_ref[m, :] + 1e-16)
        agg = acc_ref[m] * inv[:, None]
        t += jnp.dot(agg, ucat_ref[m * D:(m + 1) * D, :],
                     preferred_element_type=F32)
    z = jnp.tanh(t + c1_ref[...])
    out_ref[...] = jnp.dot(z, w2_ref[...], preferred_element_type=F32) + b2_ref[...]


def _tc_mlp(h, acc, den, u0, ucat, c1, w2, b2, blk=1024):
    grid = NPAD // blk
    return pl.pallas_call(
        _mlp_body,
        grid=(grid,),
        in_specs=[
            pl.BlockSpec((blk, D), lambda i: (i, 0)),
            pl.BlockSpec((NUM_MOTIFS, blk, D), lambda i: (0, i, 0)),
            pl.BlockSpec((NUM_MOTIFS, blk), lambda i: (0, i)),
            pl.BlockSpec((D, D), lambda i: (0, 0)),
            pl.BlockSpec((NUM_MOTIFS * D, D), lambda i: (0, 0)),
            pl.BlockSpec((1, D), lambda i: (0, 0)),
            pl.BlockSpec((D, D), lambda i: (0, 0)),
            pl.BlockSpec((1, D), lambda i: (0, 0)),
        ],
        out_specs=pl.BlockSpec((blk, D), lambda i: (i, 0)),
        out_shape=jax.ShapeDtypeStruct((NPAD, D), F32),
    )(h, acc, den, u0, ucat, c1, w2, b2)


def _sc_edges_build():
    mesh = plsc.VectorSubcoreMesh(core_axis_name="c", subcore_axis_name="s")

    @functools.partial(
        pl.kernel,
        mesh=mesh,
        compiler_params=pltpu.CompilerParams(needs_layout_passes=False,
                                             use_tc_tiling_on_sc=False),
        out_type=[
            jax.ShapeDtypeStruct((NUM_MOTIFS, NPAD, D), F32),
            jax.ShapeDtypeStruct((NUM_MOTIFS * NPAD,), F32),
        ],
        scratch_types=[
            pltpu.VMEM((N,), F32),
            pltpu.VMEM((N,), F32),
            pltpu.VMEM((2, CH), jnp.int32),
            pltpu.VMEM((2, CH), jnp.int32),
            pltpu.VMEM((2, CH), F32),
            pltpu.VMEM((2, CH, D), F32),
            pltpu.VMEM_SHARED((NPAD, D), F32),
            pltpu.VMEM_SHARED((NPAD,), F32),
            pltpu.SemaphoreType.DMA,
            pltpu.SemaphoreType.DMA,
            pltpu.SemaphoreType.DMA,
            pltpu.SemaphoreType.DMA,
        ],
    )
    def sc_edges(s_hbm, edges_hbm, h_hbm, acc_out, den_out,
                 ssrc_v, sdst_v, src_v, dsti_v, ee_v, rows_v,
                 acc_sh, den_sh, sem_g0, sem_g1, sem_s0, sem_s1):
        c = lax.axis_index("c")
        s = lax.axis_index("s")
        base = s * ROWS_PER_TILE

        for mm in range(2):
            m = 2 * c + mm

            def _zrow(r, _):
                for k in range(D // 16):
                    rows_v[0, r, pl.ds(16 * k, 16)] = jnp.zeros((16,), F32)
                return 0
            lax.fori_loop(0, CH, _zrow, 0, unroll=False)
            for t in range(CH // 16):
                ee_v[0, pl.ds(16 * t, 16)] = jnp.zeros((16,), F32)
            for j in range(ROWS_PER_TILE // CH):
                pltpu.sync_copy(rows_v.at[0],
                                acc_sh.at[pl.ds(base + CH * j, CH)])
            for j in range(ROWS_PER_TILE // CH):
                pltpu.sync_copy(ee_v.at[0],
                                den_sh.at[pl.ds(base + CH * j, CH)])

            soff = pl.multiple_of(2 * m * NPAD, NPAD)
            pltpu.sync_copy(s_hbm.at[pl.ds(soff, N)], ssrc_v)
            pltpu.sync_copy(s_hbm.at[pl.ds(soff + NPAD, N)], sdst_v)

            iota16b = lax.iota(jnp.int32, 16)

            zvec16 = jnp.zeros((16,), jnp.int32)

            def _redmax(ref):
                def body(i, carry):
                    return jnp.maximum(carry, ref[pl.ds(16 * i, 16)])
                mv = lax.fori_loop(0, N // 16, body,
                                   jnp.full((16,), -3e38, F32))
                for sh in (1, 2, 4, 8):
                    ee_v[0, pl.ds(0, 16)] = mv
                    pv = plsc.load_gather(ee_v, [zvec16, iota16b ^ sh])
                    mv = jnp.maximum(mv, pv)
                return mv[0]
            bsum = _redmax(ssrc_v) + _redmax(sdst_v)
            bound = jnp.maximum(bsum, 0.2 * bsum)

            plsc.subcore_barrier()

            def _compute(b, bound):
                for t in range(CH // 16):
                    iv = src_v[b, pl.ds(16 * t, 16)]
                    dv = dsti_v[b, pl.ds(16 * t, 16)]
                    a = plsc.load_gather(ssrc_v, [iv])
                    bb = plsc.load_gather(sdst_v, [dv])
                    sv = a + bb
                    ev = jnp.maximum(sv, 0.2 * sv) - bound
                    ee_v[b, pl.ds(16 * t, 16)] = jnp.exp(ev)

                def _srow(t, _):
                    wv = ee_v[b, pl.ds(16 * t, 16)]
                    rbase = 16 * t
                    for j in range(16):
                        w = wv[j]
                        for k in range(D // 16):
                            rows_v[b, rbase + j, pl.ds(16 * k, 16)] = (
                                rows_v[b, rbase + j, pl.ds(16 * k, 16)] * w)
                    return 0
                lax.fori_loop(0, CH // 16, _srow, 0, unroll=False)

            def _pair(i2, _):
                cidA = s + 16 * (2 * i2)
                cidB = s + 16 * (2 * i2 + 1)
                offA = pl.multiple_of(2 * E * m + cidA * CH, CH)
                offB = pl.multiple_of(2 * E * m + cidB * CH, CH)
                pltpu.sync_copy(edges_hbm.at[pl.ds(offA, CH)], src_v.at[0])
                gA = pltpu.async_copy(h_hbm.at[src_v.at[0]], rows_v.at[0],
                                      sem_g0)
                pltpu.sync_copy(edges_hbm.at[pl.ds(offB, CH)], src_v.at[1])
                gB = pltpu.async_copy(h_hbm.at[src_v.at[1]], rows_v.at[1],
                                      sem_g1)
                pltpu.sync_copy(edges_hbm.at[pl.ds(offA + E, CH)],
                                dsti_v.at[0])
                pltpu.sync_copy(edges_hbm.at[pl.ds(offB + E, CH)],
                                dsti_v.at[1])
                gA.wait()
                _compute(0, bound)
                sA1 = pltpu.async_copy(rows_v.at[0],
                                       acc_sh.at[dsti_v.at[0]], sem_s0,
                                       add=True)
                sA2 = pltpu.async_copy(ee_v.at[0],
                                       den_sh.at[dsti_v.at[0]], sem_s0,
                                       add=True)
                gB.wait()
                _compute(1, bound)
                sB1 = pltpu.async_copy(rows_v.at[1],
                                       acc_sh.at[dsti_v.at[1]], sem_s1,
                                       add=True)
                sB2 = pltpu.async_copy(ee_v.at[1],
                                       den_sh.at[dsti_v.at[1]], sem_s1,
                                       add=True)
                sA1.wait()
                sA2.wait()
                sB1.wait()
                sB2.wait()
                return 0
            lax.fori_loop(0, CPT // 2, _pair, 0, unroll=False)

            plsc.subcore_barrier()

            pltpu.sync_copy(acc_sh.at[pl.ds(base, ROWS_PER_TILE)],
                            acc_out.at[m, pl.ds(base, ROWS_PER_TILE)])
            doff = pl.multiple_of(m * NPAD + base, ROWS_PER_TILE)
            pltpu.sync_copy(den_sh.at[pl.ds(base, ROWS_PER_TILE)],
                            den_out.at[pl.ds(doff, ROWS_PER_TILE)])

            plsc.subcore_barrier()

    return sc_edges


_SC_EDGES = None


def _get_sc_edges():
    global _SC_EDGES
    if _SC_EDGES is None:
        _SC_EDGES = _sc_edges_build()
    return _SC_EDGES


def kernel(x, edges0, edges1, edges2, edges3, params):
    edges = jnp.stack([edges0.astype(jnp.int32), edges1.astype(jnp.int32),
                       edges2.astype(jnp.int32),
                       edges3.astype(jnp.int32)]).reshape(-1)
    h = jnp.pad(x.astype(F32), ((0, NPAD - N), (0, 0)))
    sc_edges = _get_sc_edges()

    for l in range(2):
        vcols = []
        ucols = []
        c1 = params[f"mlpb1_{l}"]
        for m in range(NUM_MOTIFS):
            W = params[f"W_{l}_{m}"]
            vcols.append(W @ params[f"as_{l}_{m}"])
            vcols.append(W @ params[f"ad_{l}_{m}"])
            w1m = params[f"mlpW1_{l}"][(m + 1) * D:(m + 2) * D, :]
            ucols.append(W @ w1m)
            c1 = c1 + params[f"b_{l}_{m}"] @ w1m
        vsd = jnp.stack(vcols, axis=1)
        ucat = jnp.concatenate(ucols, axis=0)
        u0 = params[f"mlpW1_{l}"][:D, :]
        w2 = params[f"mlpW2_{l}"]
        b2 = params[f"mlpb2_{l}"][None, :]
        c1 = c1[None, :]

        s_t = _tc_scores(h, vsd).reshape(-1)
        acc, den = sc_edges(s_t, edges, h)
        h = _tc_mlp(h, acc, den.reshape(NUM_MOTIFS, NPAD), u0, ucat, c1,
                    w2, b2)

    return h[:N]
